# baseline (speedup 1.0000x reference)
import functools

import jax
import jax.numpy as jnp
from jax import lax
from jax.experimental import pallas as pl
from jax.experimental.pallas import tpu as pltpu
from jax.experimental.pallas import tpu_sc as plsc

N = 10000
E = 320000
BN_EPS = 1e-5

NC = 2
NS = 16
NW = NC * NS
EPT = E // NW
CH = 2000
NCHUNK = EPT // CH
N_PAD = 10240
RPT = N_PAD // NS

_f32 = jnp.float32
_i32 = jnp.int32


def _sc_conv_body(h_hbm, src_hbm, dst_hbm, w_hbm, out_hbm,
                  accum, src_v, dst_v, w_v, rows_v, sem):
    c = lax.axis_index("c")
    s = lax.axis_index("s")
    wid = c * NS + s

    def _zero(i, carry):
        rows_v[i, :] = jnp.zeros((16,), _f32)
        return carry
    lax.fori_loop(0, RPT, _zero, 0)
    pltpu.sync_copy(rows_v.at[pl.ds(0, RPT)], accum.at[pl.ds(s * RPT, RPT)])
    plsc.subcore_barrier()

    for ci in range(NCHUNK):
        base = wid * EPT + ci * CH
        pltpu.sync_copy(src_hbm.at[pl.ds(base, CH)], src_v)
        pltpu.sync_copy(dst_hbm.at[pl.ds(base, CH)], dst_v)
        pltpu.sync_copy(w_hbm.at[pl.ds(base, CH)], w_v)
        pltpu.async_copy(h_hbm.at[src_v], rows_v, sem).wait()

        def _scale(i, carry):
            w16 = w_v[pl.ds(i * 16, 16)]
            for u in range(16):
                e = i * 16 + u
                rows_v[e, :] = rows_v[e, :] * w16[u]
            return carry
        lax.fori_loop(0, CH // 16, _scale, 0)

        pltpu.sync_copy(rows_v, accum.at[dst_v], add=True)

    plsc.subcore_barrier()
    pltpu.sync_copy(accum.at[pl.ds(s * RPT, RPT)],
                    out_hbm.at[c, pl.ds(s * RPT, RPT)])


def _sc_conv(h, src, dst, w):
    mesh = plsc.VectorSubcoreMesh(core_axis_name="c", subcore_axis_name="s",
                                  num_cores=NC, num_subcores=NS)
    fn = pl.kernel(
        _sc_conv_body,
        out_type=jax.ShapeDtypeStruct((NC, N_PAD, 16), _f32),
        mesh=mesh,
        scratch_types=[
            pltpu.VMEM_SHARED((N_PAD, 16), _f32),
            pltpu.VMEM((CH,), _i32),
            pltpu.VMEM((CH,), _i32),
            pltpu.VMEM((CH,), _f32),
            pltpu.VMEM((CH, 16), _f32),
            pltpu.SemaphoreType.DMA,
        ],
        compiler_params=pltpu.CompilerParams(use_tc_tiling_on_sc=False),
    )
    return fn(h, src, dst, w)


def _mm_body(x_ref, k_ref, o_ref):
    o_ref[...] = jnp.dot(x_ref[...], k_ref[...],
                         preferred_element_type=_f32,
                         precision=lax.Precision.HIGHEST)


def _tc_matmul(x, k):
    return pl.pallas_call(
        _mm_body,
        out_shape=jax.ShapeDtypeStruct((x.shape[0], k.shape[1]), _f32),
    )(x, k)


def _combine_body(p_ref, b_ref, o_ref):
    o_ref[...] = jnp.maximum(p_ref[0, :N] + p_ref[1, :N] + b_ref[...], 0.0)


def _tc_combine_bias_relu(p, b):
    return pl.pallas_call(
        _combine_body,
        out_shape=jax.ShapeDtypeStruct((N, 16), _f32),
    )(p, b.reshape(1, 16))


def _block2_body(p_ref, k2_ref, b2_ref, g_ref, o_ref, k3_ref, h3_ref):
    s2 = p_ref[0, :N] + p_ref[1, :N]
    m = jnp.mean(s2, axis=0, keepdims=True)
    sc2 = s2 - m
    g = lax.dot_general(sc2, sc2, (((0,), (0,)), ((), ())),
                        precision=lax.Precision.HIGHEST)
    k2 = k2_ref[...]
    var = jnp.sum(k2 * jnp.dot(g / N, k2,
                               precision=lax.Precision.HIGHEST),
                  axis=0, keepdims=True)
    inv = lax.rsqrt(var + BN_EPS)
    yc = jnp.dot(sc2, k2, precision=lax.Precision.HIGHEST)
    x2 = jnp.maximum(yc * inv * g_ref[...] + o_ref[...], 0.0)
    h3_ref[...] = jnp.dot(x2, k3_ref[...],
                          precision=lax.Precision.HIGHEST)


def _tc_block2(p, k2, b2, bn_scale, bn_offset, k3):
    return pl.pallas_call(
        _block2_body,
        out_shape=jax.ShapeDtypeStruct((N, 16), _f32),
    )(p, k2, b2.reshape(1, -1), bn_scale.reshape(1, -1),
      bn_offset.reshape(1, -1), k3)


def _final_body(p_ref, k4_ref, b4_ref, o_ref):
    s4 = p_ref[0, :N] + p_ref[1, :N]
    o_ref[...] = jnp.dot(s4, k4_ref[...],
                         preferred_element_type=_f32,
                         precision=lax.Precision.HIGHEST) + b4_ref[...]


def _tc_final(p, k4, b4):
    return pl.pallas_call(
        _final_body,
        out_shape=jax.ShapeDtypeStruct((N, k4.shape[1]), _f32),
    )(p, k4, b4.reshape(1, -1))


def kernel(edge_index, edge_weight, node_features, k1, b1, k2, b2,
           bn_scale, bn_offset, k3, b3, k4, b4):
    src = edge_index[0].astype(_i32)
    dst = edge_index[1].astype(_i32)
    w = edge_weight.astype(_f32)

    h1 = _tc_matmul(node_features, k1)
    p1 = _sc_conv(h1, src, dst, w)
    z1 = _tc_combine_bias_relu(p1, b1)
    p2 = _sc_conv(z1, src, dst, w)
    h3 = _tc_block2(p2, k2, b2, bn_scale, bn_offset, k3)
    p3 = _sc_conv(h3, src, dst, w)
    z3 = _tc_combine_bias_relu(p3, b3)
    p4 = _sc_conv(z3, src, dst, w)
    return _tc_final(p4, k4, b4)


# speedup vs baseline: 1.2453x; 1.2453x over previous
import functools

import jax
import jax.numpy as jnp
from jax import lax
from jax.experimental import pallas as pl
from jax.experimental.pallas import tpu as pltpu
from jax.experimental.pallas import tpu_sc as plsc

N = 10000
E = 320000
BN_EPS = 1e-5

NC = 2
NS = 16
NW = NC * NS
EPT = E // NW
CH = 2000
NCHUNK = EPT // CH
N_PAD = 10240
RPT = N_PAD // NS

_f32 = jnp.float32
_i32 = jnp.int32


def _sc_conv_body(h_hbm, src_hbm, dst_hbm, w_hbm, out_hbm,
                  accum,
                  src0, src1, src2, dst0, dst1, dst2, w0, w1, w2,
                  rows0, rows1, rows2,
                  g0, g1, g2, s0, s1, s2):
    c = lax.axis_index("c")
    s = lax.axis_index("s")
    wid = c * NS + s
    srcs = (src0, src1, src2)
    dsts = (dst0, dst1, dst2)
    ws = (w0, w1, w2)
    rows = (rows0, rows1, rows2)
    gsem = (g0, g1, g2)
    ssem = (s0, s1, s2)

    def _zero(i, carry):
        rows0[i, :] = jnp.zeros((16,), _f32)
        return carry
    lax.fori_loop(0, RPT, _zero, 0)
    pltpu.sync_copy(rows0.at[pl.ds(0, RPT)], accum.at[pl.ds(s * RPT, RPT)])
    plsc.subcore_barrier()

    def _load_idx(b, ci):
        base = wid * EPT + ci * CH
        pltpu.sync_copy(src_hbm.at[pl.ds(base, CH)], srcs[b])
        pltpu.sync_copy(dst_hbm.at[pl.ds(base, CH)], dsts[b])
        pltpu.sync_copy(w_hbm.at[pl.ds(base, CH)], ws[b])

    def _scale(b):
        rv, wv = rows[b], ws[b]

        def _body(i, carry):
            w16 = wv[pl.ds(i * 16, 16)]
            for u in range(16):
                e = i * 16 + u
                rv[e, :] = rv[e, :] * w16[u]
            return carry
        lax.fori_loop(0, CH // 16, _body, 0)

    gd = {}
    sd = {}
    for j in range(2):
        _load_idx(j, j)
        gd[j] = pltpu.async_copy(h_hbm.at[srcs[j]], rows[j], gsem[j])
    for j in range(NCHUNK):
        b = j % 3
        gd[j].wait()
        _scale(b)
        sd[j] = pltpu.async_copy(rows[b], accum.at[dsts[b]], ssem[b],
                                 add=True)
        nj = j + 2
        if nj < NCHUNK:
            nb = nj % 3
            if j >= 1:
                sd[j - 1].wait()
            _load_idx(nb, nj)
            gd[nj] = pltpu.async_copy(h_hbm.at[srcs[nb]], rows[nb],
                                      gsem[nb])
    for j in range(max(0, NCHUNK - 3), NCHUNK):
        sd[j].wait()

    plsc.subcore_barrier()
    pltpu.sync_copy(accum.at[pl.ds(s * RPT, RPT)],
                    out_hbm.at[c, pl.ds(s * RPT, RPT)])


def _sc_conv(h, src, dst, w):
    mesh = plsc.VectorSubcoreMesh(core_axis_name="c", subcore_axis_name="s",
                                  num_cores=NC, num_subcores=NS)
    fn = pl.kernel(
        _sc_conv_body,
        out_type=jax.ShapeDtypeStruct((NC, N_PAD, 16), _f32),
        mesh=mesh,
        scratch_types=(
            [pltpu.VMEM_SHARED((N_PAD, 16), _f32)]
            + [pltpu.VMEM((CH,), _i32)] * 3
            + [pltpu.VMEM((CH,), _i32)] * 3
            + [pltpu.VMEM((CH,), _f32)] * 3
            + [pltpu.VMEM((CH, 16), _f32)] * 3
            + [pltpu.SemaphoreType.DMA] * 6
        ),
        compiler_params=pltpu.CompilerParams(use_tc_tiling_on_sc=False),
    )
    return fn(h, src, dst, w)


def _mm_body(x_ref, k_ref, o_ref):
    o_ref[...] = jnp.dot(x_ref[...], k_ref[...],
                         preferred_element_type=_f32,
                         precision=lax.Precision.HIGHEST)


def _tc_matmul(x, k):
    return pl.pallas_call(
        _mm_body,
        out_shape=jax.ShapeDtypeStruct((x.shape[0], k.shape[1]), _f32),
    )(x, k)


def _combine_body(p_ref, b_ref, o_ref):
    o_ref[...] = jnp.maximum(p_ref[0, :N] + p_ref[1, :N] + b_ref[...], 0.0)


def _tc_combine_bias_relu(p, b):
    return pl.pallas_call(
        _combine_body,
        out_shape=jax.ShapeDtypeStruct((N, 16), _f32),
    )(p, b.reshape(1, 16))


def _block2_body(p_ref, k2_ref, b2_ref, g_ref, o_ref, k3_ref, h3_ref):
    s2 = p_ref[0, :N] + p_ref[1, :N]
    m = jnp.mean(s2, axis=0, keepdims=True)
    sc2 = s2 - m
    g = lax.dot_general(sc2, sc2, (((0,), (0,)), ((), ())),
                        precision=lax.Precision.HIGHEST)
    k2 = k2_ref[...]
    var = jnp.sum(k2 * jnp.dot(g / N, k2,
                               precision=lax.Precision.HIGHEST),
                  axis=0, keepdims=True)
    inv = lax.rsqrt(var + BN_EPS)
    yc = jnp.dot(sc2, k2, precision=lax.Precision.HIGHEST)
    x2 = jnp.maximum(yc * inv * g_ref[...] + o_ref[...], 0.0)
    h3_ref[...] = jnp.dot(x2, k3_ref[...],
                          precision=lax.Precision.HIGHEST)


def _tc_block2(p, k2, b2, bn_scale, bn_offset, k3):
    return pl.pallas_call(
        _block2_body,
        out_shape=jax.ShapeDtypeStruct((N, 16), _f32),
    )(p, k2, b2.reshape(1, -1), bn_scale.reshape(1, -1),
      bn_offset.reshape(1, -1), k3)


def _final_body(p_ref, k4_ref, b4_ref, o_ref):
    s4 = p_ref[0, :N] + p_ref[1, :N]
    o_ref[...] = jnp.dot(s4, k4_ref[...],
                         preferred_element_type=_f32,
                         precision=lax.Precision.HIGHEST) + b4_ref[...]


def _tc_final(p, k4, b4):
    return pl.pallas_call(
        _final_body,
        out_shape=jax.ShapeDtypeStruct((N, k4.shape[1]), _f32),
    )(p, k4, b4.reshape(1, -1))


def kernel(edge_index, edge_weight, node_features, k1, b1, k2, b2,
           bn_scale, bn_offset, k3, b3, k4, b4):
    src = edge_index[0].astype(_i32)
    dst = edge_index[1].astype(_i32)
    w = edge_weight.astype(_f32)

    h1 = _tc_matmul(node_features, k1)
    p1 = _sc_conv(h1, src, dst, w)
    z1 = _tc_combine_bias_relu(p1, b1)
    p2 = _sc_conv(z1, src, dst, w)
    h3 = _tc_block2(p2, k2, b2, bn_scale, bn_offset, k3)
    p3 = _sc_conv(h3, src, dst, w)
    z3 = _tc_combine_bias_relu(p3, b3)
    p4 = _sc_conv(z3, src, dst, w)
    return _tc_final(p4, k4, b4)


# speedup vs baseline: 1.2570x; 1.0094x over previous
import functools

import jax
import jax.numpy as jnp
from jax import lax
from jax.experimental import pallas as pl
from jax.experimental.pallas import tpu as pltpu
from jax.experimental.pallas import tpu_sc as plsc

N = 10000
E = 320000
BN_EPS = 1e-5

NC = 2
NS = 16
NW = NC * NS
EPT = E // NW
CH = 2000
NCHUNK = EPT // CH
N_PAD = 10240
RPT = N_PAD // NS

_f32 = jnp.float32
_i32 = jnp.int32


def _sc_conv_body(h_hbm, src_hbm, dst_hbm, w_hbm, out_hbm,
                  accum,
                  src0, src1, src2, dst0, dst1, dst2, w0, w1, w2,
                  rows0, rows1, rows2,
                  g0, g1, g2, s0, s1, s2):
    c = lax.axis_index("c")
    s = lax.axis_index("s")
    wid = c * NS + s
    srcs = (src0, src1, src2)
    dsts = (dst0, dst1, dst2)
    ws = (w0, w1, w2)
    rows = (rows0, rows1, rows2)
    gsem = (g0, g1, g2)
    ssem = (s0, s1, s2)

    def _zero(i, carry):
        rows0[i, :] = jnp.zeros((16,), _f32)
        return carry
    lax.fori_loop(0, RPT, _zero, 0)
    pltpu.sync_copy(rows0.at[pl.ds(0, RPT)], accum.at[pl.ds(s * RPT, RPT)])
    plsc.subcore_barrier()

    def _load_idx(b, ci):
        base = wid * EPT + ci * CH
        pltpu.sync_copy(src_hbm.at[pl.ds(base, CH)], srcs[b])
        pltpu.sync_copy(dst_hbm.at[pl.ds(base, CH)], dsts[b])
        pltpu.sync_copy(w_hbm.at[pl.ds(base, CH)], ws[b])

    def _scale(b):
        rv, wv = rows[b], ws[b]

        def _body(i, carry):
            wa = wv[pl.ds(i * 32, 16)]
            wb = wv[pl.ds(i * 32 + 16, 16)]
            for u in range(16):
                e = i * 32 + u
                rv[e, :] = rv[e, :] * wa[u]
                rv[e + 16, :] = rv[e + 16, :] * wb[u]
            return carry
        lax.fori_loop(0, CH // 32, _body, 0)

    gd = {}
    sd = {}
    for j in range(2):
        _load_idx(j, j)
        gd[j] = pltpu.async_copy(h_hbm.at[srcs[j]], rows[j], gsem[j])
    for j in range(NCHUNK):
        b = j % 3
        gd[j].wait()
        _scale(b)
        sd[j] = pltpu.async_copy(rows[b], accum.at[dsts[b]], ssem[b],
                                 add=True)
        nj = j + 2
        if nj < NCHUNK:
            nb = nj % 3
            if j >= 1:
                sd[j - 1].wait()
            _load_idx(nb, nj)
            gd[nj] = pltpu.async_copy(h_hbm.at[srcs[nb]], rows[nb],
                                      gsem[nb])
    for j in range(max(0, NCHUNK - 3), NCHUNK):
        sd[j].wait()

    plsc.subcore_barrier()
    pltpu.sync_copy(accum.at[pl.ds(s * RPT, RPT)],
                    out_hbm.at[c, pl.ds(s * RPT, RPT)])


def _sc_conv(h, src, dst, w):
    mesh = plsc.VectorSubcoreMesh(core_axis_name="c", subcore_axis_name="s",
                                  num_cores=NC, num_subcores=NS)
    fn = pl.kernel(
        _sc_conv_body,
        out_type=jax.ShapeDtypeStruct((NC, N_PAD, 16), _f32),
        mesh=mesh,
        scratch_types=(
            [pltpu.VMEM_SHARED((N_PAD, 16), _f32)]
            + [pltpu.VMEM((CH,), _i32)] * 3
            + [pltpu.VMEM((CH,), _i32)] * 3
            + [pltpu.VMEM((CH,), _f32)] * 3
            + [pltpu.VMEM((CH, 16), _f32)] * 3
            + [pltpu.SemaphoreType.DMA] * 6
        ),
        compiler_params=pltpu.CompilerParams(use_tc_tiling_on_sc=False),
    )
    return fn(h, src, dst, w)


def _mm_body(x_ref, k_ref, o_ref):
    o_ref[...] = jnp.dot(x_ref[...], k_ref[...],
                         preferred_element_type=_f32,
                         precision=lax.Precision.HIGHEST)


def _tc_matmul(x, k):
    return pl.pallas_call(
        _mm_body,
        out_shape=jax.ShapeDtypeStruct((x.shape[0], k.shape[1]), _f32),
    )(x, k)


def _combine_body(p_ref, b_ref, o_ref):
    o_ref[...] = jnp.maximum(p_ref[0, :N] + p_ref[1, :N] + b_ref[...], 0.0)


def _tc_combine_bias_relu(p, b):
    return pl.pallas_call(
        _combine_body,
        out_shape=jax.ShapeDtypeStruct((N, 16), _f32),
    )(p, b.reshape(1, 16))


def _block2_body(p_ref, k2_ref, b2_ref, g_ref, o_ref, k3_ref, h3_ref):
    s2 = p_ref[0, :N] + p_ref[1, :N]
    m = jnp.mean(s2, axis=0, keepdims=True)
    sc2 = s2 - m
    g = lax.dot_general(sc2, sc2, (((0,), (0,)), ((), ())),
                        precision=lax.Precision.HIGHEST)
    k2 = k2_ref[...]
    var = jnp.sum(k2 * jnp.dot(g / N, k2,
                               precision=lax.Precision.HIGHEST),
                  axis=0, keepdims=True)
    inv = lax.rsqrt(var + BN_EPS)
    yc = jnp.dot(sc2, k2, precision=lax.Precision.HIGHEST)
    x2 = jnp.maximum(yc * inv * g_ref[...] + o_ref[...], 0.0)
    h3_ref[...] = jnp.dot(x2, k3_ref[...],
                          precision=lax.Precision.HIGHEST)


def _tc_block2(p, k2, b2, bn_scale, bn_offset, k3):
    return pl.pallas_call(
        _block2_body,
        out_shape=jax.ShapeDtypeStruct((N, 16), _f32),
    )(p, k2, b2.reshape(1, -1), bn_scale.reshape(1, -1),
      bn_offset.reshape(1, -1), k3)


def _final_body(p_ref, k4_ref, b4_ref, o_ref):
    s4 = p_ref[0, :N] + p_ref[1, :N]
    o_ref[...] = jnp.dot(s4, k4_ref[...],
                         preferred_element_type=_f32,
                         precision=lax.Precision.HIGHEST) + b4_ref[...]


def _tc_final(p, k4, b4):
    return pl.pallas_call(
        _final_body,
        out_shape=jax.ShapeDtypeStruct((N, k4.shape[1]), _f32),
    )(p, k4, b4.reshape(1, -1))


def kernel(edge_index, edge_weight, node_features, k1, b1, k2, b2,
           bn_scale, bn_offset, k3, b3, k4, b4):
    src = edge_index[0].astype(_i32)
    dst = edge_index[1].astype(_i32)
    w = edge_weight.astype(_f32)

    h1 = _tc_matmul(node_features, k1)
    p1 = _sc_conv(h1, src, dst, w)
    z1 = _tc_combine_bias_relu(p1, b1)
    p2 = _sc_conv(z1, src, dst, w)
    h3 = _tc_block2(p2, k2, b2, bn_scale, bn_offset, k3)
    p3 = _sc_conv(h3, src, dst, w)
    z3 = _tc_combine_bias_relu(p3, b3)
    p4 = _sc_conv(z3, src, dst, w)
    return _tc_final(p4, k4, b4)
